# baseline (speedup 1.0000x reference)
import functools

import numpy as np
import jax
import jax.numpy as jnp
from jax import lax
from jax.experimental import pallas as pl
from jax.experimental.pallas import tpu as pltpu
from jax.experimental.pallas import tpu_sc as plsc

EPS = 1e-6
_BA = 64
_BC = 32


def _cumsum_lanes(x, n, tri):
    b = x.shape[0]
    chunks = []
    carry = jnp.zeros((b, 1), jnp.float32)
    for c in range(n // 128):
        yc = x[:, c * 128 : (c + 1) * 128]
        sc = jnp.dot(yc, tri, preferred_element_type=jnp.float32) + carry
        chunks.append(sc)
        carry = sc[:, 127:128]
    return jnp.concatenate(chunks, axis=-1)


def _grid_weights(t):
    dx = t[:, 1:] - t[:, :-1]
    z1 = jnp.zeros((1, 1), jnp.float32)
    dxa = jnp.concatenate([dx, z1], axis=1)
    dxb = jnp.concatenate([z1, dx], axis=1)
    return dxa, dxa + dxb


def _cdf_of(x, dxa, wsum, tri):
    n = x.shape[-1]
    a = jnp.abs(x) + EPS
    s = _cumsum_lanes(a * wsum, n, tri)
    tot = s[:, n - 1 : n]
    return (s - a * dxa) / tot, a, 2.0 / tot


def _body_a(obs_ref, t_ref, tri_ref, cdf_ref):
    dxa, wsum = _grid_weights(t_ref[...])
    cdf, _, _ = _cdf_of(obs_ref[...], dxa, wsum, tri_ref[...])
    cdf_ref[...] = cdf


def _body_c(tr_ref, q_ref, t_ref, p_ref, ainvt_ref, tri_ref, out_ref, *, nt, kk, bb):
    t = t_ref[...]
    p = p_ref[...]
    dxa, wsum = _grid_weights(t)

    q = q_ref[...]
    hk = p[:, 1:] - p[:, :-1]
    dq = (q[:, 1:] - q[:, :-1]) / hk
    zb = jnp.zeros((bb, 1), jnp.float32)
    rhs = jnp.concatenate([zb, 6.0 * (dq[:, 1:] - dq[:, :-1]), zb], axis=1)
    m = jnp.dot(rhs, ainvt_ref[...], preferred_element_type=jnp.float32)

    cdf, a, inv_z = _cdf_of(tr_ref[...], dxa, wsum, tri_ref[...])
    i = jnp.clip((cdf * (kk - 1)).astype(jnp.int32), 0, kk - 2)
    pb = jnp.broadcast_to(p, (bb, kk))
    x0 = jnp.take_along_axis(pb, i, axis=1)
    x1 = jnp.take_along_axis(pb, i + 1, axis=1)
    y0 = jnp.take_along_axis(q, i, axis=1)
    y1 = jnp.take_along_axis(q, i + 1, axis=1)
    m0 = jnp.take_along_axis(m, i, axis=1)
    m1 = jnp.take_along_axis(m, i + 1, axis=1)
    hl = x1 - x0
    al = x1 - cdf
    bl = cdf - x0
    inv_h = 1.0 / hl
    qv = (m0 * al * al * al + m1 * bl * bl * bl) * (inv_h * (1.0 / 6.0)) + (
        y0 * inv_h - m0 * hl * (1.0 / 6.0)
    ) * al + (y1 * inv_h - m1 * hl * (1.0 / 6.0)) * bl

    d = t - qv
    integ = jnp.sum(d * d * a * (0.5 * wsum) * inv_z, axis=1)
    integ2 = jnp.sum(integ, axis=0, keepdims=True)[None, :]

    @pl.when(pl.program_id(0) == 0)
    def _():
        out_ref[...] = jnp.zeros((1, 1), jnp.float32)

    out_ref[...] += integ2


def _sc_searchsorted(cdf, t, p, nw):
    n, nt = cdf.shape
    kk = p.shape[0]
    rows = n // nw
    try:
        nc = plsc.get_sparse_core_info().num_cores
    except AttributeError:
        nc = 2
    mesh = plsc.VectorSubcoreMesh(core_axis_name="c", subcore_axis_name="s")

    rg = 16
    nch = kk // 16

    @functools.partial(
        pl.kernel,
        mesh=mesh,
        compiler_params=pltpu.CompilerParams(needs_layout_passes=False),
        out_type=jax.ShapeDtypeStruct((n, kk), jnp.float32),
        scratch_types=[
            pltpu.VMEM((rg, nt), jnp.float32),
            pltpu.VMEM((nt,), jnp.float32),
            pltpu.VMEM((kk,), jnp.float32),
            pltpu.VMEM((rg, kk), jnp.float32),
        ],
    )
    def body(cdf_hbm, t_hbm, p_hbm, q_hbm, buf_v, t_v, p_v, q_v):
        wid = lax.axis_index("s") * nc + lax.axis_index("c")
        pltpu.sync_copy(t_hbm, t_v)
        pltpu.sync_copy(p_hbm, p_v)

        def group_loop(g, carry):
            base = wid * rows + g * rg
            pltpu.sync_copy(cdf_hbm.at[pl.ds(base, rg)], buf_v)

            def row_fn(r, carry2):
                rvec = jnp.full((16,), r, jnp.int32)
                pvals = [p_v[pl.ds(c * 16, 16)] for c in range(nch)]
                poss = [jnp.zeros((16,), jnp.int32) for _ in range(nch)]
                step = nt // 2
                while step >= 1:
                    for c in range(nch):
                        cand = poss[c] + step
                        vals = plsc.load_gather(buf_v, [rvec, cand - 1])
                        poss[c] = jnp.where(vals < pvals[c], cand, poss[c])
                    step //= 2
                for c in range(nch):
                    idx = jnp.minimum(poss[c], nt - 1)
                    q_v.at[r][pl.ds(c * 16, 16)] = plsc.load_gather(t_v, [idx])
                return carry2

            lax.fori_loop(0, rg, row_fn, 0)
            pltpu.sync_copy(q_v, q_hbm.at[pl.ds(base, rg)])
            return carry

        lax.fori_loop(0, rows // rg, group_loop, 0)

    return body(cdf, t, p)


def _spline_inv_t(kk):
    h = 1.0 / (kk - 1)
    A = np.zeros((kk, kk), np.float64)
    A[0, 0] = 1.0
    A[kk - 1, kk - 1] = 1.0
    r = np.arange(1, kk - 1)
    A[r, r - 1] = h
    A[r, r] = 4.0 * h
    A[r, r + 1] = h
    return np.linalg.inv(A).T.astype(np.float32)


def kernel(traces, obs_data, t, p):
    n, nt = traces.shape
    kk = p.shape[0]
    t2 = t.reshape(1, nt)
    p2 = p.reshape(1, kk)
    ainvt = jnp.asarray(_spline_inv_t(kk))
    tri = jnp.asarray(np.triu(np.ones((128, 128), np.float32)))

    obs_cdf = pl.pallas_call(
        _body_a,
        grid=(n // _BA,),
        in_specs=[
            pl.BlockSpec((_BA, nt), lambda i: (i, 0)),
            pl.BlockSpec((1, nt), lambda i: (0, 0)),
            pl.BlockSpec((128, 128), lambda i: (0, 0)),
        ],
        out_specs=pl.BlockSpec((_BA, nt), lambda i: (i, 0)),
        out_shape=jax.ShapeDtypeStruct((n, nt), jnp.float32),
    )(obs_data, t2, tri)

    q = _sc_searchsorted(obs_cdf, t, p, 32)

    body_c = functools.partial(_body_c, nt=nt, kk=kk, bb=_BC)
    out = pl.pallas_call(
        body_c,
        grid=(n // _BC,),
        in_specs=[
            pl.BlockSpec((_BC, nt), lambda i: (i, 0)),
            pl.BlockSpec((_BC, kk), lambda i: (i, 0)),
            pl.BlockSpec((1, nt), lambda i: (0, 0)),
            pl.BlockSpec((1, kk), lambda i: (0, 0)),
            pl.BlockSpec((kk, kk), lambda i: (0, 0)),
            pl.BlockSpec((128, 128), lambda i: (0, 0)),
        ],
        out_specs=pl.BlockSpec((1, 1), lambda i: (0, 0)),
        out_shape=jax.ShapeDtypeStruct((1, 1), jnp.float32),
    )(traces, q, t2, p2, ainvt, tri)
    return out[0, 0]


# speedup vs baseline: 1.5716x; 1.5716x over previous
import functools

import numpy as np
import jax
import jax.numpy as jnp
from jax import lax
from jax.experimental import pallas as pl
from jax.experimental.pallas import tpu as pltpu
from jax.experimental.pallas import tpu_sc as plsc

EPS = 1e-6
_BA = 128
_BC = 64


def _cumsum_lanes(x, n, tri):
    b = x.shape[0]
    chunks = []
    carry = jnp.zeros((b, 1), jnp.float32)
    for c in range(n // 128):
        yc = x[:, c * 128 : (c + 1) * 128]
        sc = jnp.dot(yc, tri, preferred_element_type=jnp.float32) + carry
        chunks.append(sc)
        carry = sc[:, 127:128]
    return jnp.concatenate(chunks, axis=-1)


def _grid_weights(t):
    dx = t[:, 1:] - t[:, :-1]
    z1 = jnp.zeros((1, 1), jnp.float32)
    dxa = jnp.concatenate([dx, z1], axis=1)
    dxb = jnp.concatenate([z1, dx], axis=1)
    return dxa, dxa + dxb


def _cdf_of(x, dxa, wsum, tri):
    n = x.shape[-1]
    a = jnp.abs(x) + EPS
    s = _cumsum_lanes(a * wsum, n, tri)
    tot = s[:, n - 1 : n]
    rtot = 1.0 / tot
    return (s - a * dxa) * rtot, a, 2.0 * rtot


def _body_a(obs_ref, t_ref, tri_ref, cdf_ref):
    dxa, wsum = _grid_weights(t_ref[...])
    cdf, _, _ = _cdf_of(obs_ref[...], dxa, wsum, tri_ref[...])
    cdf_ref[...] = cdf


def _body_c(tr_ref, q_ref, t_ref, p_ref, ainvt_ref, tri_ref, out_ref, *, nt, kk, bb):
    t = t_ref[...]
    p = p_ref[...]
    dxa, wsum = _grid_weights(t)

    q = q_ref[...]
    hk = p[:, 1:] - p[:, :-1]
    dq = (q[:, 1:] - q[:, :-1]) / hk
    zb = jnp.zeros((bb, 1), jnp.float32)
    rhs = jnp.concatenate([zb, 6.0 * (dq[:, 1:] - dq[:, :-1]), zb], axis=1)
    m = jnp.dot(rhs, ainvt_ref[...], preferred_element_type=jnp.float32)

    cdf, a, inv_z = _cdf_of(tr_ref[...], dxa, wsum, tri_ref[...])
    i = jnp.clip((cdf * (kk - 1)).astype(jnp.int32), 0, kk - 2)
    y0 = jnp.take_along_axis(q, i, axis=1)
    y1 = jnp.take_along_axis(q, i + 1, axis=1)
    m0 = jnp.take_along_axis(m, i, axis=1)
    m1 = jnp.take_along_axis(m, i + 1, axis=1)
    hc = 1.0 / (kk - 1)
    inv_h = float(kk - 1)
    x0 = i.astype(jnp.float32) * hc
    al = x0 + hc - cdf
    bl = cdf - x0
    qv = (m0 * al * al * al + m1 * bl * bl * bl) * (inv_h * (1.0 / 6.0)) + (
        y0 * inv_h - m0 * (hc * (1.0 / 6.0))
    ) * al + (y1 * inv_h - m1 * (hc * (1.0 / 6.0))) * bl

    d = t - qv
    integ = jnp.sum(d * d * a * (0.5 * wsum) * inv_z, axis=1)
    integ2 = jnp.sum(integ, axis=0, keepdims=True)[None, :]

    @pl.when(pl.program_id(0) == 0)
    def _():
        out_ref[...] = jnp.zeros((1, 1), jnp.float32)

    out_ref[...] += integ2


def _sc_searchsorted(cdf, t, p, nw):
    n, nt = cdf.shape
    kk = p.shape[0]
    rows = n // nw
    try:
        nc = plsc.get_sparse_core_info().num_cores
    except AttributeError:
        nc = 2
    mesh = plsc.VectorSubcoreMesh(core_axis_name="c", subcore_axis_name="s")

    rg = 16
    nch = kk // 16

    @functools.partial(
        pl.kernel,
        mesh=mesh,
        compiler_params=pltpu.CompilerParams(needs_layout_passes=False),
        out_type=jax.ShapeDtypeStruct((n, kk), jnp.float32),
        scratch_types=[
            pltpu.VMEM((rg, nt), jnp.float32),
            pltpu.VMEM((nt,), jnp.float32),
            pltpu.VMEM((kk,), jnp.float32),
            pltpu.VMEM((rg, kk), jnp.float32),
        ],
    )
    def body(cdf_hbm, t_hbm, p_hbm, q_hbm, buf_v, t_v, p_v, q_v):
        wid = lax.axis_index("s") * nc + lax.axis_index("c")
        pltpu.sync_copy(t_hbm, t_v)
        pltpu.sync_copy(p_hbm, p_v)

        def group_loop(g, carry):
            base = wid * rows + g * rg
            pltpu.sync_copy(cdf_hbm.at[pl.ds(base, rg)], buf_v)

            def row_fn(r, carry2):
                rvec = jnp.full((16,), r, jnp.int32)
                pvals = [p_v[pl.ds(c * 16, 16)] for c in range(nch)]
                poss = [jnp.zeros((16,), jnp.int32) for _ in range(nch)]
                step = nt // 2
                while step >= 1:
                    for c in range(nch):
                        cand = poss[c] + step
                        vals = plsc.load_gather(buf_v, [rvec, cand - 1])
                        poss[c] = jnp.where(vals < pvals[c], cand, poss[c])
                    step //= 2
                for c in range(nch):
                    idx = jnp.minimum(poss[c], nt - 1)
                    q_v.at[r][pl.ds(c * 16, 16)] = plsc.load_gather(t_v, [idx])
                return carry2

            lax.fori_loop(0, rg, row_fn, 0)
            pltpu.sync_copy(q_v, q_hbm.at[pl.ds(base, rg)])
            return carry

        lax.fori_loop(0, rows // rg, group_loop, 0)

    return body(cdf, t, p)


def _spline_inv_t(kk):
    h = 1.0 / (kk - 1)
    A = np.zeros((kk, kk), np.float64)
    A[0, 0] = 1.0
    A[kk - 1, kk - 1] = 1.0
    r = np.arange(1, kk - 1)
    A[r, r - 1] = h
    A[r, r] = 4.0 * h
    A[r, r + 1] = h
    return np.linalg.inv(A).T.astype(np.float32)


def kernel(traces, obs_data, t, p):
    n, nt = traces.shape
    kk = p.shape[0]
    t2 = t.reshape(1, nt)
    p2 = p.reshape(1, kk)
    ainvt = jnp.asarray(_spline_inv_t(kk))
    tri = jnp.asarray(np.triu(np.ones((128, 128), np.float32)))

    obs_cdf = pl.pallas_call(
        _body_a,
        grid=(n // _BA,),
        in_specs=[
            pl.BlockSpec((_BA, nt), lambda i: (i, 0)),
            pl.BlockSpec((1, nt), lambda i: (0, 0)),
            pl.BlockSpec((128, 128), lambda i: (0, 0)),
        ],
        out_specs=pl.BlockSpec((_BA, nt), lambda i: (i, 0)),
        out_shape=jax.ShapeDtypeStruct((n, nt), jnp.float32),
    )(obs_data, t2, tri)

    q = _sc_searchsorted(obs_cdf, t, p, 32)

    body_c = functools.partial(_body_c, nt=nt, kk=kk, bb=_BC)
    out = pl.pallas_call(
        body_c,
        grid=(n // _BC,),
        in_specs=[
            pl.BlockSpec((_BC, nt), lambda i: (i, 0)),
            pl.BlockSpec((_BC, kk), lambda i: (i, 0)),
            pl.BlockSpec((1, nt), lambda i: (0, 0)),
            pl.BlockSpec((1, kk), lambda i: (0, 0)),
            pl.BlockSpec((kk, kk), lambda i: (0, 0)),
            pl.BlockSpec((128, 128), lambda i: (0, 0)),
        ],
        out_specs=pl.BlockSpec((1, 1), lambda i: (0, 0)),
        out_shape=jax.ShapeDtypeStruct((1, 1), jnp.float32),
    )(traces, q, t2, p2, ainvt, tri)
    return out[0, 0]


# speedup vs baseline: 2.3103x; 1.4700x over previous
import functools

import numpy as np
import jax
import jax.numpy as jnp
from jax import lax
from jax.experimental import pallas as pl
from jax.experimental.pallas import tpu as pltpu
from jax.experimental.pallas import tpu_sc as plsc

EPS = 1e-6
_BA = 512
_BC = 512


def _cumsum_lanes(x, n, tri):
    b = x.shape[0]
    chunks = []
    carry = jnp.zeros((b, 1), jnp.float32)
    for c in range(n // 128):
        yc = x[:, c * 128 : (c + 1) * 128]
        sc = jnp.dot(yc, tri, preferred_element_type=jnp.float32) + carry
        chunks.append(sc)
        carry = sc[:, 127:128]
    return jnp.concatenate(chunks, axis=-1)


def _grid_weights(t):
    dx = t[:, 1:] - t[:, :-1]
    z1 = jnp.zeros((1, 1), jnp.float32)
    dxa = jnp.concatenate([dx, z1], axis=1)
    dxb = jnp.concatenate([z1, dx], axis=1)
    return dxa, dxa + dxb


def _cdf_of(x, dxa, wsum, tri):
    n = x.shape[-1]
    a = jnp.abs(x) + EPS
    s = _cumsum_lanes(a * wsum, n, tri)
    tot = s[:, n - 1 : n]
    rtot = 1.0 / tot
    return (s - a * dxa) * rtot, a, 2.0 * rtot


def _body_a(obs_ref, t_ref, tri_ref, cdf_ref):
    dxa, wsum = _grid_weights(t_ref[...])
    cdf, _, _ = _cdf_of(obs_ref[...], dxa, wsum, tri_ref[...])
    cdf_ref[...] = cdf


def _body_c(tr_ref, q_ref, t_ref, p_ref, ainvt_ref, tri_ref, out_ref, *, nt, kk, bb):
    t = t_ref[...]
    p = p_ref[...]
    dxa, wsum = _grid_weights(t)

    q = q_ref[...]
    hk = p[:, 1:] - p[:, :-1]
    dq = (q[:, 1:] - q[:, :-1]) / hk
    zb = jnp.zeros((bb, 1), jnp.float32)
    rhs = jnp.concatenate([zb, 6.0 * (dq[:, 1:] - dq[:, :-1]), zb], axis=1)
    m = jnp.dot(rhs, ainvt_ref[...], preferred_element_type=jnp.float32)

    cdf, a, inv_z = _cdf_of(tr_ref[...], dxa, wsum, tri_ref[...])
    i = jnp.clip((cdf * (kk - 1)).astype(jnp.int32), 0, kk - 2)
    y0 = jnp.take_along_axis(q, i, axis=1)
    y1 = jnp.take_along_axis(q, i + 1, axis=1)
    m0 = jnp.take_along_axis(m, i, axis=1)
    m1 = jnp.take_along_axis(m, i + 1, axis=1)
    hc = 1.0 / (kk - 1)
    inv_h = float(kk - 1)
    x0 = i.astype(jnp.float32) * hc
    al = x0 + hc - cdf
    bl = cdf - x0
    qv = (m0 * al * al * al + m1 * bl * bl * bl) * (inv_h * (1.0 / 6.0)) + (
        y0 * inv_h - m0 * (hc * (1.0 / 6.0))
    ) * al + (y1 * inv_h - m1 * (hc * (1.0 / 6.0))) * bl

    d = t - qv
    integ = jnp.sum(d * d * a * (0.5 * wsum) * inv_z, axis=1)
    integ2 = jnp.sum(integ, axis=0, keepdims=True)[None, :]

    @pl.when(pl.program_id(0) == 0)
    def _():
        out_ref[...] = jnp.zeros((1, 1), jnp.float32)

    out_ref[...] += integ2


def _sc_searchsorted(cdf, t, p, nw):
    n, nt = cdf.shape
    kk = p.shape[0]
    rows = n // nw
    try:
        nc = plsc.get_sparse_core_info().num_cores
    except AttributeError:
        nc = 2
    mesh = plsc.VectorSubcoreMesh(core_axis_name="c", subcore_axis_name="s")

    rg = 16
    nch = kk // 16

    @functools.partial(
        pl.kernel,
        mesh=mesh,
        compiler_params=pltpu.CompilerParams(needs_layout_passes=False),
        out_type=jax.ShapeDtypeStruct((n, kk), jnp.float32),
        scratch_types=[
            pltpu.VMEM((rg, nt), jnp.float32),
            pltpu.VMEM((nt,), jnp.float32),
            pltpu.VMEM((kk,), jnp.float32),
            pltpu.VMEM((rg, kk), jnp.float32),
        ],
    )
    def body(cdf_hbm, t_hbm, p_hbm, q_hbm, buf_v, t_v, p_v, q_v):
        wid = lax.axis_index("s") * nc + lax.axis_index("c")
        pltpu.sync_copy(t_hbm, t_v)
        pltpu.sync_copy(p_hbm, p_v)

        def group_loop(g, carry):
            base = wid * rows + g * rg
            pltpu.sync_copy(cdf_hbm.at[pl.ds(base, rg)], buf_v)

            def row_fn(r, carry2):
                rvec = jnp.full((16,), r, jnp.int32)
                pvals = [p_v[pl.ds(c * 16, 16)] for c in range(nch)]
                poss = [jnp.zeros((16,), jnp.int32) for _ in range(nch)]
                step = nt // 2
                while step >= 1:
                    for c in range(nch):
                        cand = poss[c] + step
                        vals = plsc.load_gather(buf_v, [rvec, cand - 1])
                        poss[c] = jnp.where(vals < pvals[c], cand, poss[c])
                    step //= 2
                for c in range(nch):
                    idx = jnp.minimum(poss[c], nt - 1)
                    q_v.at[r][pl.ds(c * 16, 16)] = plsc.load_gather(t_v, [idx])
                return carry2

            lax.fori_loop(0, rg, row_fn, 0)
            pltpu.sync_copy(q_v, q_hbm.at[pl.ds(base, rg)])
            return carry

        lax.fori_loop(0, rows // rg, group_loop, 0)

    return body(cdf, t, p)


def _spline_inv_t(kk):
    h = 1.0 / (kk - 1)
    A = np.zeros((kk, kk), np.float64)
    A[0, 0] = 1.0
    A[kk - 1, kk - 1] = 1.0
    r = np.arange(1, kk - 1)
    A[r, r - 1] = h
    A[r, r] = 4.0 * h
    A[r, r + 1] = h
    return np.linalg.inv(A).T.astype(np.float32)


def kernel(traces, obs_data, t, p):
    n, nt = traces.shape
    kk = p.shape[0]
    t2 = t.reshape(1, nt)
    p2 = p.reshape(1, kk)
    ainvt = jnp.asarray(_spline_inv_t(kk))
    tri = jnp.asarray(np.triu(np.ones((128, 128), np.float32)))

    obs_cdf = pl.pallas_call(
        _body_a,
        grid=(n // _BA,),
        in_specs=[
            pl.BlockSpec((_BA, nt), lambda i: (i, 0)),
            pl.BlockSpec((1, nt), lambda i: (0, 0)),
            pl.BlockSpec((128, 128), lambda i: (0, 0)),
        ],
        out_specs=pl.BlockSpec((_BA, nt), lambda i: (i, 0)),
        out_shape=jax.ShapeDtypeStruct((n, nt), jnp.float32),
    )(obs_data, t2, tri)

    q = _sc_searchsorted(obs_cdf, t, p, 32)

    body_c = functools.partial(_body_c, nt=nt, kk=kk, bb=_BC)
    out = pl.pallas_call(
        body_c,
        grid=(n // _BC,),
        in_specs=[
            pl.BlockSpec((_BC, nt), lambda i: (i, 0)),
            pl.BlockSpec((_BC, kk), lambda i: (i, 0)),
            pl.BlockSpec((1, nt), lambda i: (0, 0)),
            pl.BlockSpec((1, kk), lambda i: (0, 0)),
            pl.BlockSpec((kk, kk), lambda i: (0, 0)),
            pl.BlockSpec((128, 128), lambda i: (0, 0)),
        ],
        out_specs=pl.BlockSpec((1, 1), lambda i: (0, 0)),
        out_shape=jax.ShapeDtypeStruct((1, 1), jnp.float32),
    )(traces, q, t2, p2, ainvt, tri)
    return out[0, 0]


# speedup vs baseline: 2.8868x; 1.2496x over previous
import functools

import numpy as np
import jax
import jax.numpy as jnp
from jax import lax
from jax.experimental import pallas as pl
from jax.experimental.pallas import tpu as pltpu
from jax.experimental.pallas import tpu_sc as plsc

EPS = 1e-6
_BA = 512
_BC = 512


def _cumsum_lanes(x, n, tri):
    b = x.shape[0]
    chunks = []
    carry = jnp.zeros((b, 1), jnp.float32)
    for c in range(n // 128):
        yc = x[:, c * 128 : (c + 1) * 128]
        sc = jnp.dot(yc, tri, preferred_element_type=jnp.float32) + carry
        chunks.append(sc)
        carry = sc[:, 127:128]
    return jnp.concatenate(chunks, axis=-1)


def _grid_weights(t):
    dx = t[:, 1:] - t[:, :-1]
    z1 = jnp.zeros((1, 1), jnp.float32)
    dxa = jnp.concatenate([dx, z1], axis=1)
    dxb = jnp.concatenate([z1, dx], axis=1)
    return dxa, dxa + dxb


def _cdf_of(x, dxa, wsum, tri):
    n = x.shape[-1]
    a = jnp.abs(x) + EPS
    s = _cumsum_lanes(a * wsum, n, tri)
    tot = s[:, n - 1 : n]
    rtot = 1.0 / tot
    return (s - a * dxa) * rtot, a, 2.0 * rtot


def _body_a(obs_ref, t_ref, tri_ref, cdf_ref):
    dxa, wsum = _grid_weights(t_ref[...])
    cdf, _, _ = _cdf_of(obs_ref[...], dxa, wsum, tri_ref[...])
    cdf_ref[...] = cdf


def _body_c(prev_ref, tr_ref, q_ref, t_ref, p_ref, ainvt_ref, tri_ref, out_ref, *, nt, kk, bb):
    t = t_ref[...]
    p = p_ref[...]
    dxa, wsum = _grid_weights(t)

    q = q_ref[...]
    hk = p[:, 1:] - p[:, :-1]
    dq = (q[:, 1:] - q[:, :-1]) / hk
    zb = jnp.zeros((bb, 1), jnp.float32)
    rhs = jnp.concatenate([zb, 6.0 * (dq[:, 1:] - dq[:, :-1]), zb], axis=1)
    m = jnp.dot(rhs, ainvt_ref[...], preferred_element_type=jnp.float32)

    cdf, a, inv_z = _cdf_of(tr_ref[...], dxa, wsum, tri_ref[...])
    i = jnp.clip((cdf * (kk - 1)).astype(jnp.int32), 0, kk - 2)
    y0 = jnp.take_along_axis(q, i, axis=1)
    y1 = jnp.take_along_axis(q, i + 1, axis=1)
    m0 = jnp.take_along_axis(m, i, axis=1)
    m1 = jnp.take_along_axis(m, i + 1, axis=1)
    hc = 1.0 / (kk - 1)
    inv_h = float(kk - 1)
    x0 = i.astype(jnp.float32) * hc
    al = x0 + hc - cdf
    bl = cdf - x0
    qv = (m0 * al * al * al + m1 * bl * bl * bl) * (inv_h * (1.0 / 6.0)) + (
        y0 * inv_h - m0 * (hc * (1.0 / 6.0))
    ) * al + (y1 * inv_h - m1 * (hc * (1.0 / 6.0))) * bl

    d = t - qv
    integ = jnp.sum(d * d * a * (0.5 * wsum) * inv_z, axis=1)
    integ2 = jnp.sum(integ, axis=0, keepdims=True)[None, :]

    @pl.when(pl.program_id(0) == 0)
    def _():
        out_ref[...] = prev_ref[...]

    out_ref[...] += integ2


def _sc_searchsorted(cdf, t, p, nw):
    n, nt = cdf.shape
    kk = p.shape[0]
    rows = n // nw
    try:
        nc = plsc.get_sparse_core_info().num_cores
    except AttributeError:
        nc = 2
    mesh = plsc.VectorSubcoreMesh(core_axis_name="c", subcore_axis_name="s")

    rg = 16
    nch = kk // 16

    @functools.partial(
        pl.kernel,
        mesh=mesh,
        compiler_params=pltpu.CompilerParams(needs_layout_passes=False),
        out_type=jax.ShapeDtypeStruct((n, kk), jnp.float32),
        scratch_types=[
            pltpu.VMEM((rg, nt), jnp.float32),
            pltpu.VMEM((nt,), jnp.float32),
            pltpu.VMEM((kk,), jnp.float32),
            pltpu.VMEM((rg, kk), jnp.float32),
        ],
    )
    def body(cdf_hbm, t_hbm, p_hbm, q_hbm, buf_v, t_v, p_v, q_v):
        wid = lax.axis_index("s") * nc + lax.axis_index("c")
        pltpu.sync_copy(t_hbm, t_v)
        pltpu.sync_copy(p_hbm, p_v)

        def group_loop(g, carry):
            base = wid * rows + g * rg
            pltpu.sync_copy(cdf_hbm.at[pl.ds(base, rg)], buf_v)

            def row_fn(r, carry2):
                rvec = jnp.full((16,), r, jnp.int32)
                pvals = [p_v[pl.ds(c * 16, 16)] for c in range(nch)]
                poss = [jnp.zeros((16,), jnp.int32) for _ in range(nch)]
                step = nt // 2
                while step >= 1:
                    for c in range(nch):
                        cand = poss[c] + step
                        vals = plsc.load_gather(buf_v, [rvec, cand - 1])
                        poss[c] = jnp.where(vals < pvals[c], cand, poss[c])
                    step //= 2
                for c in range(nch):
                    idx = jnp.minimum(poss[c], nt - 1)
                    q_v.at[r][pl.ds(c * 16, 16)] = plsc.load_gather(t_v, [idx])
                return carry2

            lax.fori_loop(0, rg, row_fn, 0)
            pltpu.sync_copy(q_v, q_hbm.at[pl.ds(base, rg)])
            return carry

        lax.fori_loop(0, rows // rg, group_loop, 0)

    return body(cdf, t, p)


def _spline_inv_t(kk):
    h = 1.0 / (kk - 1)
    A = np.zeros((kk, kk), np.float64)
    A[0, 0] = 1.0
    A[kk - 1, kk - 1] = 1.0
    r = np.arange(1, kk - 1)
    A[r, r - 1] = h
    A[r, r] = 4.0 * h
    A[r, r + 1] = h
    return np.linalg.inv(A).T.astype(np.float32)


def kernel(traces, obs_data, t, p):
    n, nt = traces.shape
    kk = p.shape[0]
    t2 = t.reshape(1, nt)
    p2 = p.reshape(1, kk)
    ainvt = jnp.asarray(_spline_inv_t(kk))
    tri = jnp.asarray(np.triu(np.ones((128, 128), np.float32)))

    nsl = 2
    half = n // nsl
    sa = half // _BA
    sc_steps = half // _BC

    def pass_a(h):
        return pl.pallas_call(
            _body_a,
            grid=(sa,),
            in_specs=[
                pl.BlockSpec((_BA, nt), lambda i, h=h: (i + h * sa, 0)),
                pl.BlockSpec((1, nt), lambda i: (0, 0)),
                pl.BlockSpec((128, 128), lambda i: (0, 0)),
            ],
            out_specs=pl.BlockSpec((_BA, nt), lambda i: (i, 0)),
            out_shape=jax.ShapeDtypeStruct((half, nt), jnp.float32),
        )(obs_data, t2, tri)

    body_c = functools.partial(_body_c, nt=nt, kk=kk, bb=_BC)

    def pass_c(h, prev, q):
        return pl.pallas_call(
            body_c,
            grid=(sc_steps,),
            in_specs=[
                pl.BlockSpec((1, 1), lambda i: (0, 0)),
                pl.BlockSpec((_BC, nt), lambda i, h=h: (i + h * sc_steps, 0)),
                pl.BlockSpec((_BC, kk), lambda i: (i, 0)),
                pl.BlockSpec((1, nt), lambda i: (0, 0)),
                pl.BlockSpec((1, kk), lambda i: (0, 0)),
                pl.BlockSpec((kk, kk), lambda i: (0, 0)),
                pl.BlockSpec((128, 128), lambda i: (0, 0)),
            ],
            out_specs=pl.BlockSpec((1, 1), lambda i: (0, 0)),
            out_shape=jax.ShapeDtypeStruct((1, 1), jnp.float32),
        )(prev, traces, q, t2, p2, ainvt, tri)

    qs = []
    for h in range(nsl):
        cdf_h = pass_a(h)
        qs.append(_sc_searchsorted(cdf_h, t, p, 32))
    loss = jnp.zeros((1, 1), jnp.float32)
    for h in range(nsl):
        loss = pass_c(h, loss, qs[h])
    return loss[0, 0]


# speedup vs baseline: 3.2232x; 1.1165x over previous
import functools

import numpy as np
import jax
import jax.numpy as jnp
from jax import lax
from jax.experimental import pallas as pl
from jax.experimental.pallas import tpu as pltpu
from jax.experimental.pallas import tpu_sc as plsc

EPS = 1e-6
_BA = 512
_BC = 512


def _cumsum_lanes(x, n, tri):
    b = x.shape[0]
    chunks = []
    carry = jnp.zeros((b, 1), jnp.float32)
    for c in range(n // 128):
        yc = x[:, c * 128 : (c + 1) * 128]
        sc = jnp.dot(yc, tri, preferred_element_type=jnp.float32) + carry
        chunks.append(sc)
        carry = sc[:, 127:128]
    return jnp.concatenate(chunks, axis=-1)


def _grid_weights(t):
    dx = t[:, 1:] - t[:, :-1]
    z1 = jnp.zeros((1, 1), jnp.float32)
    dxa = jnp.concatenate([dx, z1], axis=1)
    dxb = jnp.concatenate([z1, dx], axis=1)
    return dxa, dxa + dxb


def _cdf_of(x, dxa, wsum, tri):
    n = x.shape[-1]
    a = jnp.abs(x) + EPS
    s = _cumsum_lanes(a * wsum, n, tri)
    tot = s[:, n - 1 : n]
    rtot = 1.0 / tot
    return (s - a * dxa) * rtot, a, 2.0 * rtot


def _body_a(obs_ref, t_ref, tri_ref, cdf_ref):
    dxa, wsum = _grid_weights(t_ref[...])
    cdf, _, _ = _cdf_of(obs_ref[...], dxa, wsum, tri_ref[...])
    cdf_ref[...] = cdf


def _body_c(prev_ref, tr_ref, q_ref, t_ref, p_ref, ainvt_ref, tri_ref, out_ref, *, nt, kk, bb):
    t = t_ref[...]
    p = p_ref[...]
    dxa, wsum = _grid_weights(t)

    q = q_ref[...]
    hk = p[:, 1:] - p[:, :-1]
    dq = (q[:, 1:] - q[:, :-1]) / hk
    zb = jnp.zeros((bb, 1), jnp.float32)
    rhs = jnp.concatenate([zb, 6.0 * (dq[:, 1:] - dq[:, :-1]), zb], axis=1)
    m = jnp.dot(rhs, ainvt_ref[...], preferred_element_type=jnp.float32)

    cdf, a, inv_z = _cdf_of(tr_ref[...], dxa, wsum, tri_ref[...])
    i = jnp.clip((cdf * (kk - 1)).astype(jnp.int32), 0, kk - 2)
    y0 = jnp.take_along_axis(q, i, axis=1)
    y1 = jnp.take_along_axis(q, i + 1, axis=1)
    m0 = jnp.take_along_axis(m, i, axis=1)
    m1 = jnp.take_along_axis(m, i + 1, axis=1)
    hc = 1.0 / (kk - 1)
    inv_h = float(kk - 1)
    x0 = i.astype(jnp.float32) * hc
    al = x0 + hc - cdf
    bl = cdf - x0
    qv = (m0 * al * al * al + m1 * bl * bl * bl) * (inv_h * (1.0 / 6.0)) + (
        y0 * inv_h - m0 * (hc * (1.0 / 6.0))
    ) * al + (y1 * inv_h - m1 * (hc * (1.0 / 6.0))) * bl

    d = t - qv
    integ = jnp.sum(d * d * a * (0.5 * wsum) * inv_z, axis=1)
    integ2 = jnp.sum(integ, axis=0, keepdims=True)[None, :]

    @pl.when(pl.program_id(0) == 0)
    def _():
        out_ref[...] = prev_ref[...]

    out_ref[...] += integ2


def _sc_searchsorted(cdf, t, p, nw):
    n, nt = cdf.shape
    kk = p.shape[0]
    rows = n // nw
    try:
        nc = plsc.get_sparse_core_info().num_cores
    except AttributeError:
        nc = 2
    mesh = plsc.VectorSubcoreMesh(core_axis_name="c", subcore_axis_name="s")

    rg = 16
    nch = kk // 16

    @functools.partial(
        pl.kernel,
        mesh=mesh,
        compiler_params=pltpu.CompilerParams(needs_layout_passes=False),
        out_type=jax.ShapeDtypeStruct((n, kk), jnp.float32),
        scratch_types=[
            pltpu.VMEM((rg, nt), jnp.float32),
            pltpu.VMEM((nt,), jnp.float32),
            pltpu.VMEM((kk,), jnp.float32),
            pltpu.VMEM((rg, kk), jnp.float32),
        ],
    )
    def body(cdf_hbm, t_hbm, p_hbm, q_hbm, buf_v, t_v, p_v, q_v):
        wid = lax.axis_index("s") * nc + lax.axis_index("c")
        pltpu.sync_copy(t_hbm, t_v)
        pltpu.sync_copy(p_hbm, p_v)

        def group_loop(g, carry):
            base = wid * rows + g * rg
            pltpu.sync_copy(cdf_hbm.at[pl.ds(base, rg)], buf_v)

            def row_fn(r, carry2):
                rvec = jnp.full((16,), r, jnp.int32)
                pvals = [p_v[pl.ds(c * 16, 16)] for c in range(nch)]
                poss = [jnp.zeros((16,), jnp.int32) for _ in range(nch)]
                step = nt // 2
                while step >= 1:
                    for c in range(nch):
                        cand = poss[c] + step
                        vals = plsc.load_gather(buf_v, [rvec, cand - 1])
                        poss[c] = jnp.where(vals < pvals[c], cand, poss[c])
                    step //= 2
                for c in range(nch):
                    idx = jnp.minimum(poss[c], nt - 1)
                    q_v.at[r][pl.ds(c * 16, 16)] = plsc.load_gather(t_v, [idx])
                return carry2

            lax.fori_loop(0, rg, row_fn, 0)
            pltpu.sync_copy(q_v, q_hbm.at[pl.ds(base, rg)])
            return carry

        lax.fori_loop(0, rows // rg, group_loop, 0)

    return body(cdf, t, p)


def _spline_inv_t(kk):
    h = 1.0 / (kk - 1)
    A = np.zeros((kk, kk), np.float64)
    A[0, 0] = 1.0
    A[kk - 1, kk - 1] = 1.0
    r = np.arange(1, kk - 1)
    A[r, r - 1] = h
    A[r, r] = 4.0 * h
    A[r, r + 1] = h
    return np.linalg.inv(A).T.astype(np.float32)


def kernel(traces, obs_data, t, p):
    n, nt = traces.shape
    kk = p.shape[0]
    t2 = t.reshape(1, nt)
    p2 = p.reshape(1, kk)
    ainvt = jnp.asarray(_spline_inv_t(kk))
    tri = jnp.asarray(np.triu(np.ones((128, 128), np.float32)))

    nsl = 4
    half = n // nsl
    sa = half // _BA
    sc_steps = half // _BC

    def pass_a(h):
        return pl.pallas_call(
            _body_a,
            grid=(sa,),
            in_specs=[
                pl.BlockSpec((_BA, nt), lambda i, h=h: (i + h * sa, 0)),
                pl.BlockSpec((1, nt), lambda i: (0, 0)),
                pl.BlockSpec((128, 128), lambda i: (0, 0)),
            ],
            out_specs=pl.BlockSpec((_BA, nt), lambda i: (i, 0)),
            out_shape=jax.ShapeDtypeStruct((half, nt), jnp.float32),
        )(obs_data, t2, tri)

    body_c = functools.partial(_body_c, nt=nt, kk=kk, bb=_BC)

    def pass_c(h, prev, q):
        return pl.pallas_call(
            body_c,
            grid=(sc_steps,),
            in_specs=[
                pl.BlockSpec((1, 1), lambda i: (0, 0)),
                pl.BlockSpec((_BC, nt), lambda i, h=h: (i + h * sc_steps, 0)),
                pl.BlockSpec((_BC, kk), lambda i: (i, 0)),
                pl.BlockSpec((1, nt), lambda i: (0, 0)),
                pl.BlockSpec((1, kk), lambda i: (0, 0)),
                pl.BlockSpec((kk, kk), lambda i: (0, 0)),
                pl.BlockSpec((128, 128), lambda i: (0, 0)),
            ],
            out_specs=pl.BlockSpec((1, 1), lambda i: (0, 0)),
            out_shape=jax.ShapeDtypeStruct((1, 1), jnp.float32),
        )(prev, traces, q, t2, p2, ainvt, tri)

    qs = []
    for h in range(nsl):
        cdf_h = pass_a(h)
        qs.append(_sc_searchsorted(cdf_h, t, p, 32))
    loss = jnp.zeros((1, 1), jnp.float32)
    for h in range(nsl):
        loss = pass_c(h, loss, qs[h])
    return loss[0, 0]
